# baseline (speedup 1.0000x reference)
import functools

import jax
import jax.numpy as jnp
from jax import lax
from jax.experimental import pallas as pl
from jax.experimental.pallas import tpu as pltpu
from jax.experimental.pallas import tpu_sc as plsc

N = 10000
E = 320000
D = 128
R = 6
NC, NS, LANES = 2, 16, 16
NW = NC * NS
EPT = E // NW
CH = 80
NCHUNK = EPT // CH
ROWS_PT = N // NS
WIN = 640
CW = 16
HD = 64
NPAD = 10240
GCH = 80
GPT = NPAD // NW
GNC = GPT // GCH
BM = 1000


def _sc_mesh():
  return plsc.VectorSubcoreMesh(
      core_axis_name="c", subcore_axis_name="s", num_cores=NC,
      num_subcores=NS)


def _embed_body(ut, uid2, it, iid2, hu_out, hi_out, idxv, rows, sem, osem):
  wid = lax.axis_index("s") * NC + lax.axis_index("c")
  tables = ((ut, uid2, hu_out), (it, iid2, hi_out))
  for t, (tbl, idx2, out) in enumerate(tables):
    pltpu.sync_copy(idx2.at[wid], idxv.at[t])
  gathers = []
  for t, (tbl, idx2, out) in enumerate(tables):
    for c in range(GNC):
      gathers.append(pltpu.async_copy(
          tbl.at[idxv.at[t].at[c]], rows.at[t].at[pl.ds(c * GCH, GCH)], sem))
  for g in gathers:
    g.wait()
  outcp = []
  for t, (tbl, idx2, out) in enumerate(tables):
    outcp.append(pltpu.async_copy(rows.at[t],
                                  out.at[pl.ds(wid * GPT, GPT)], osem))
  for o in outcp:
    o.wait()


def _embed_call(user_table, uid2, item_table, iid2):
  f = pl.kernel(
      _embed_body,
      out_type=tuple(jax.ShapeDtypeStruct((NPAD, D), jnp.float32)
                     for _ in range(2)),
      mesh=_sc_mesh(),
      compiler_params=pltpu.CompilerParams(use_tc_tiling_on_sc=False),
      scratch_types=[
          pltpu.VMEM((2, GNC, GCH), jnp.int32),
          pltpu.VMEM((2, GPT, D), jnp.float32),
          pltpu.SemaphoreType.DMA,
          pltpu.SemaphoreType.DMA,
      ],
  )
  return f(user_table, uid2, item_table, iid2)


def _msg_table_body(emb_ref, fid_ref, ftab_ref, wr_ref,
                    hr_lo_ref, hr_hi_ref, h_ref):
  nfeat = ftab_ref.shape[0]
  onehot = (fid_ref[...] == lax.broadcasted_iota(jnp.int32, (1, nfeat), 1)
            ).astype(jnp.float32)
  feat = lax.dot_general(onehot, ftab_ref[...], (((1,), (0,)), ((), ())),
                         preferred_element_type=jnp.float32,
                         precision=lax.Precision.HIGHEST)
  h = emb_ref[...] + feat
  h_ref[...] = h
  w = wr_ref[0]
  hr = lax.dot_general(h, w, (((1,), (1,)), ((), ())),
                       preferred_element_type=jnp.float32)
  hr_lo_ref[0] = hr[:, :HD]
  hr_hi_ref[0] = hr[:, HD:]


def _msg_table_call(emb, fid, ftab, wr):
  grid = (R, N // BM)
  nfeat = ftab.shape[0]
  half_spec = pl.BlockSpec((1, BM, HD), lambda r, m: (r, m, 0))
  hr_lo, hr_hi, h = pl.pallas_call(
      _msg_table_body,
      grid=grid,
      in_specs=[
          pl.BlockSpec((BM, D), lambda r, m: (m, 0)),
          pl.BlockSpec((BM, 1), lambda r, m: (m, 0)),
          pl.BlockSpec((nfeat, D), lambda r, m: (0, 0)),
          pl.BlockSpec((1, D, D), lambda r, m: (r, 0, 0)),
      ],
      out_specs=[
          half_spec,
          half_spec,
          pl.BlockSpec((BM, D), lambda r, m: (m, 0)),
      ],
      out_shape=[
          jax.ShapeDtypeStruct((R, N, HD), jnp.float32),
          jax.ShapeDtypeStruct((R, N, HD), jnp.float32),
          jax.ShapeDtypeStruct((N, D), jnp.float32),
      ],
  )(emb, fid.astype(jnp.int32).reshape(N, 1), ftab, wr)
  return hr_lo.reshape(R * N, HD), hr_hi.reshape(R * N, HD), h


def _edge_body(hr_lo, hr_hi, src2, dst2, rat2, msum_out, cnt_out,
               src_v, dst_v, rat_v, gix_v, rows, zrows, ones_v, zb,
               shared, cnt_sh, gsem0, gsem1, csem):
  cid = lax.axis_index("c")
  sid = lax.axis_index("s")
  wid = sid * NC + cid
  zero16 = jnp.zeros((LANES,), jnp.float32)
  ones16 = jnp.ones((LANES,), jnp.float32)

  pltpu.sync_copy(src2.at[wid], src_v)
  pltpu.sync_copy(dst2.at[wid], dst_v)
  pltpu.sync_copy(rat2.at[wid], rat_v)

  def gix_chunk(c, _):
    def gix_vec(j, _):
      g = rat_v[c, pl.ds(j * LANES, LANES)] * N + src_v[c, pl.ds(j * LANES, LANES)]
      gix_v[c, pl.ds(j * LANES, LANES)] = g
      return 0
    return lax.fori_loop(0, CH // LANES, gix_vec, 0)
  lax.fori_loop(0, NCHUNK, gix_chunk, 0)

  def zrow(i, _):
    def zlane(j, _):
      zrows[i, pl.ds(j * LANES, LANES)] = zero16
      return 0
    return lax.fori_loop(0, HD // LANES, zlane, 0)
  lax.fori_loop(0, CH, zrow, 0)

  def fill16(i, _):
    ones_v[i, pl.ds(0, LANES)] = ones16
    zb[i, pl.ds(0, LANES)] = zero16
    return 0
  lax.fori_loop(0, CH, fill16, 0)

  base = sid * (ROWS_PT - 1)

  for p, hr_half in enumerate((hr_lo, hr_hi)):
    for k in range(WIN // CH):
      pltpu.sync_copy(zrows, shared.at[pl.ds(base + k * CH, CH)])
      if p == 0:
        pltpu.sync_copy(zb, cnt_sh.at[pl.ds(base + k * CH, CH)])

    plsc.subcore_barrier()

    def start_gather(c, b, sem):
      pltpu.async_copy(hr_half.at[gix_v.at[c]], rows.at[b], sem)

    def finish(c, b, sem):
      pltpu.make_async_copy(hr_half.at[gix_v.at[c]], rows.at[b], sem).wait()
      pltpu.sync_copy(rows.at[b], shared.at[dst_v.at[c]], add=True)
      if p == 0:
        pltpu.make_async_copy(ones_v, cnt_sh.at[dst_v.at[c]], csem).wait()
        pltpu.async_copy(ones_v, cnt_sh.at[dst_v.at[c]], csem, add=True)

    start_gather(0, 0, gsem0)
    if p == 0:
      pltpu.async_copy(zb, cnt_sh.at[dst_v.at[0]], csem, add=True)

    def pair(k, _):
      c0 = 2 * k
      start_gather(c0 + 1, 1, gsem1)
      finish(c0, 0, gsem0)
      start_gather(c0 + 2, 0, gsem0)
      finish(c0 + 1, 1, gsem1)
      return 0
    lax.fori_loop(0, (NCHUNK - 1) // 2, pair, 0)
    finish(NCHUNK - 1, 0, gsem0)
    if p == 0:
      pltpu.make_async_copy(ones_v, cnt_sh.at[dst_v.at[0]], csem).wait()

    plsc.subcore_barrier()

    pltpu.sync_copy(shared.at[pl.ds(base, WIN)],
                    msum_out.at[p].at[cid].at[pl.ds(base, WIN)])
    if p == 0:
      pltpu.sync_copy(cnt_sh.at[pl.ds(base, WIN)],
                      cnt_out.at[cid].at[pl.ds(base, WIN)])
      plsc.subcore_barrier()


def _edge_call(hr_lo, hr_hi, src2, dst2, rat2):
  f = pl.kernel(
      _edge_body,
      out_type=(
          jax.ShapeDtypeStruct((2, NC, N, HD), jnp.float32),
          jax.ShapeDtypeStruct((NC, N, CW), jnp.float32),
      ),
      mesh=_sc_mesh(),
      compiler_params=pltpu.CompilerParams(use_tc_tiling_on_sc=False),
      scratch_types=[
          pltpu.VMEM((NCHUNK, CH), jnp.int32),
          pltpu.VMEM((NCHUNK, CH), jnp.int32),
          pltpu.VMEM((NCHUNK, CH), jnp.int32),
          pltpu.VMEM((NCHUNK, CH), jnp.int32),
          pltpu.VMEM((2, CH, HD), jnp.float32),
          pltpu.VMEM((CH, HD), jnp.float32),
          pltpu.VMEM((CH, CW), jnp.float32),
          pltpu.VMEM((CH, CW), jnp.float32),
          pltpu.VMEM_SHARED((N, HD), jnp.float32),
          pltpu.VMEM_SHARED((N, CW), jnp.float32),
          pltpu.SemaphoreType.DMA,
          pltpu.SemaphoreType.DMA,
          pltpu.SemaphoreType.DMA,
      ],
  )
  return f(hr_lo, hr_hi, src2, dst2, rat2)


def _tail_body(msi_ref, cnti_ref, hi_ref, wd_i, wn_i, bl_i, pv, pvb,
               msu_ref, cntu_ref, hu_ref, wd_u, wn_u, bl_u, pw, pwb,
               item_out, user_out):
  def side(ms_ref, cnt_ref, hd_ref, wd, wn, bl, p, pb, out):
    ms = jnp.concatenate([ms_ref[0, 0] + ms_ref[0, 1],
                          ms_ref[1, 0] + ms_ref[1, 1]], axis=1)
    cnt = (cnt_ref[0] + cnt_ref[1])[:, :1]
    hn = jnp.where(cnt > 0, ms / jnp.maximum(cnt, 1.0), 0.0)
    acc = lax.dot_general(hd_ref[...], wd[...], (((1,), (1,)), ((), ())),
                          preferred_element_type=jnp.float32)
    acc += lax.dot_general(hn, wn[...], (((1,), (1,)), ((), ())),
                           preferred_element_type=jnp.float32)
    nh = jnp.maximum(acc + bl[...], 0.0)
    out[...] = lax.dot_general(nh, p[...], (((1,), (1,)), ((), ())),
                               preferred_element_type=jnp.float32) + pb[...]
  side(msi_ref, cnti_ref, hi_ref, wd_i, wn_i, bl_i, pv, pvb, item_out)
  side(msu_ref, cntu_ref, hu_ref, wd_u, wn_u, bl_u, pw, pwb, user_out)


def _tail_call(msum_i, cnt_i, h_item, wl_i, bl_i, vw, vb,
               msum_u, cnt_u, h_user, wl_u, bl_u, ww, wb):
  grid = (N // BM,)
  row_spec = pl.BlockSpec((BM, D), lambda m: (m, 0))
  ms_spec = pl.BlockSpec((2, NC, BM, HD), lambda m: (0, 0, m, 0))
  cnt_spec = pl.BlockSpec((NC, BM, CW), lambda m: (0, m, 0))
  w_spec = pl.BlockSpec((D, D), lambda m: (0, 0))
  b_spec = pl.BlockSpec((1, D), lambda m: (0, 0))
  args = (msum_i, cnt_i, h_item, wl_i[:, :D], wl_i[:, D:],
          bl_i.reshape(1, D), vw, vb.reshape(1, D),
          msum_u, cnt_u, h_user, wl_u[:, :D], wl_u[:, D:],
          bl_u.reshape(1, D), ww, wb.reshape(1, D))
  specs = [ms_spec, cnt_spec, row_spec, w_spec, w_spec, b_spec, w_spec,
           b_spec] * 2
  out_item, out_user = pl.pallas_call(
      _tail_body,
      grid=grid,
      in_specs=specs,
      out_specs=[row_spec, row_spec],
      out_shape=[
          jax.ShapeDtypeStruct((N, D), jnp.float32),
          jax.ShapeDtypeStruct((N, D), jnp.float32),
      ],
  )(*args)
  return out_item, out_user


def _pad_idx(idx):
  idx = idx.astype(jnp.int32)
  pad = NPAD - idx.shape[0]
  return jnp.pad(idx, (0, pad)).reshape(NW, GNC, GCH)


def kernel(user_ids, item_ids, user_gender, item_genres, edge_user,
           edge_item, edge_rating, user_table, item_table, gender_table,
           genre_table, Wr_watched, Wl_watched_w, Wl_watched_b,
           Wr_watchedby, Wl_watchedby_w, Wl_watchedby_b,
           W_w, W_b, V_w, V_b):
  hu_raw, hi_raw = _embed_call(user_table, _pad_idx(user_ids),
                               item_table, _pad_idx(item_ids))

  hr_w_lo, hr_w_hi, h_user = _msg_table_call(hu_raw[:N], user_gender,
                                             gender_table, Wr_watched)
  hr_b_lo, hr_b_hi, h_item = _msg_table_call(hi_raw[:N], item_genres,
                                             genre_table, Wr_watchedby)

  eu2 = edge_user.astype(jnp.int32).reshape(NW, NCHUNK, CH)
  ei2 = edge_item.astype(jnp.int32).reshape(NW, NCHUNK, CH)
  er2 = edge_rating.astype(jnp.int32).reshape(NW, NCHUNK, CH)

  msum_i, cnt_i = _edge_call(hr_w_lo, hr_w_hi, eu2, ei2, er2)
  msum_u, cnt_u = _edge_call(hr_b_lo, hr_b_hi, ei2, eu2, er2)

  out_item, out_user = _tail_call(
      msum_i, cnt_i, h_item, Wl_watched_w, Wl_watched_b, V_w, V_b,
      msum_u, cnt_u, h_user, Wl_watchedby_w, Wl_watchedby_b, W_w, W_b)
  return (out_user, out_item)


# speedup vs baseline: 1.2639x; 1.2639x over previous
import functools

import jax
import jax.numpy as jnp
from jax import lax
from jax.experimental import pallas as pl
from jax.experimental.pallas import tpu as pltpu
from jax.experimental.pallas import tpu_sc as plsc

N = 10000
E = 320000
D = 128
R = 6
NC, NS, LANES = 2, 16, 16
NW = NC * NS
EPT = E // NW
CH = 80
NCHUNK = EPT // CH
ROWS_PT = N // NS
WIN = 640
CW = 16
HD = 64
NPAD = 10240
GCH = 80
GPT = NPAD // NW
GNC = GPT // GCH
BM = 1000


def _sc_mesh():
  return plsc.VectorSubcoreMesh(
      core_axis_name="c", subcore_axis_name="s", num_cores=NC,
      num_subcores=NS)


def _embed_body(ut, uid2, it, iid2, hu_out, hi_out, idxv, rows, sem, osem):
  wid = lax.axis_index("s") * NC + lax.axis_index("c")
  tables = ((ut, uid2, hu_out), (it, iid2, hi_out))
  for t, (tbl, idx2, out) in enumerate(tables):
    pltpu.sync_copy(idx2.at[wid], idxv.at[t])
  gathers = []
  for t, (tbl, idx2, out) in enumerate(tables):
    for c in range(GNC):
      gathers.append(pltpu.async_copy(
          tbl.at[idxv.at[t].at[c]], rows.at[t].at[pl.ds(c * GCH, GCH)], sem))
  for g in gathers:
    g.wait()
  outcp = []
  for t, (tbl, idx2, out) in enumerate(tables):
    outcp.append(pltpu.async_copy(rows.at[t],
                                  out.at[pl.ds(wid * GPT, GPT)], osem))
  for o in outcp:
    o.wait()


def _embed_call(user_table, uid2, item_table, iid2):
  f = pl.kernel(
      _embed_body,
      out_type=tuple(jax.ShapeDtypeStruct((NPAD, D), jnp.float32)
                     for _ in range(2)),
      mesh=_sc_mesh(),
      compiler_params=pltpu.CompilerParams(use_tc_tiling_on_sc=False),
      scratch_types=[
          pltpu.VMEM((2, GNC, GCH), jnp.int32),
          pltpu.VMEM((2, GPT, D), jnp.float32),
          pltpu.SemaphoreType.DMA,
          pltpu.SemaphoreType.DMA,
      ],
  )
  return f(user_table, uid2, item_table, iid2)


def _msg_side(emb_ref, fid_ref, ftab_ref, wr_ref, hr_ref, h_ref):
  nfeat = ftab_ref.shape[0]
  onehot = (fid_ref[...] == lax.broadcasted_iota(jnp.int32, (1, nfeat), 1)
            ).astype(jnp.float32)
  feat = lax.dot_general(onehot, ftab_ref[...], (((1,), (0,)), ((), ())),
                         preferred_element_type=jnp.float32,
                         precision=lax.Precision.HIGHEST)
  h = emb_ref[...] + feat
  h_ref[...] = h
  for r in range(R):
    hr_ref[r] = lax.dot_general(h, wr_ref[r], (((1,), (1,)), ((), ())),
                                preferred_element_type=jnp.float32)


def _msg_table_body(embu_ref, fidu_ref, gtab_ref, wrw_ref,
                    embi_ref, fidi_ref, ntab_ref, wrb_ref,
                    hrw_ref, hrb_ref, hu_ref, hi_ref):
  _msg_side(embu_ref, fidu_ref, gtab_ref, wrw_ref, hrw_ref, hu_ref)
  _msg_side(embi_ref, fidi_ref, ntab_ref, wrb_ref, hrb_ref, hi_ref)


def _msg_table_call(embu, fidu, gtab, wrw, embi, fidi, ntab, wrb):
  grid = (N // BM,)
  row_spec = pl.BlockSpec((BM, D), lambda m: (m, 0))
  fid_spec = pl.BlockSpec((BM, 1), lambda m: (m, 0))
  wr_spec = pl.BlockSpec((R, D, D), lambda m: (0, 0, 0))
  hr_spec = pl.BlockSpec((R, BM, D), lambda m: (0, m, 0))
  hr_shape = jax.ShapeDtypeStruct((R, N, D), jnp.float32)
  hrw, hrb, hu, hi = pl.pallas_call(
      _msg_table_body,
      grid=grid,
      in_specs=[
          row_spec, fid_spec,
          pl.BlockSpec((gtab.shape[0], D), lambda m: (0, 0)), wr_spec,
          row_spec, fid_spec,
          pl.BlockSpec((ntab.shape[0], D), lambda m: (0, 0)), wr_spec,
      ],
      out_specs=[hr_spec, hr_spec, row_spec, row_spec],
      out_shape=[hr_shape, hr_shape,
                 jax.ShapeDtypeStruct((N, D), jnp.float32),
                 jax.ShapeDtypeStruct((N, D), jnp.float32)],
  )(embu, fidu.astype(jnp.int32).reshape(N, 1), gtab, wrw,
    embi, fidi.astype(jnp.int32).reshape(N, 1), ntab, wrb)
  return hrw.reshape(2 * R * N, HD), hrb.reshape(2 * R * N, HD), hu, hi


def _edge_body(hr, src2, dst2, rat2, msum_out, cnt_out,
               src_v, dst_v, rat_v, gix_v, rows, zrows, ones_v, zb,
               shared, cnt_sh, gsem0, gsem1, csem):
  cid = lax.axis_index("c")
  sid = lax.axis_index("s")
  wid = sid * NC + cid
  zero16 = jnp.zeros((LANES,), jnp.float32)
  ones16 = jnp.ones((LANES,), jnp.float32)

  pltpu.sync_copy(src2.at[wid], src_v)
  pltpu.sync_copy(dst2.at[wid], dst_v)
  pltpu.sync_copy(rat2.at[wid], rat_v)

  def gix_chunk(c, _):
    def gix_vec(j, _):
      g0 = (rat_v[c, pl.ds(j * LANES, LANES)] * N
            + src_v[c, pl.ds(j * LANES, LANES)]) * 2
      gix_v[0, c, pl.ds(j * LANES, LANES)] = g0
      gix_v[1, c, pl.ds(j * LANES, LANES)] = g0 + 1
      return 0
    return lax.fori_loop(0, CH // LANES, gix_vec, 0)
  lax.fori_loop(0, NCHUNK, gix_chunk, 0)

  def zrow(i, _):
    def zlane(j, _):
      zrows[i, pl.ds(j * LANES, LANES)] = zero16
      return 0
    return lax.fori_loop(0, HD // LANES, zlane, 0)
  lax.fori_loop(0, CH, zrow, 0)

  def fill16(i, _):
    ones_v[i, pl.ds(0, LANES)] = ones16
    zb[i, pl.ds(0, LANES)] = zero16
    return 0
  lax.fori_loop(0, CH, fill16, 0)

  base = sid * (ROWS_PT - 1)

  for p in range(2):
    for k in range(WIN // CH):
      pltpu.sync_copy(zrows, shared.at[pl.ds(base + k * CH, CH)])
      if p == 0:
        pltpu.sync_copy(zb, cnt_sh.at[pl.ds(base + k * CH, CH)])

    plsc.subcore_barrier()

    def start_gather(c, b, sem):
      pltpu.async_copy(hr.at[gix_v.at[p].at[c]], rows.at[b], sem)

    def finish(c, b, sem):
      pltpu.make_async_copy(hr.at[gix_v.at[p].at[c]], rows.at[b], sem).wait()
      pltpu.sync_copy(rows.at[b], shared.at[dst_v.at[c]], add=True)
      if p == 0:
        pltpu.make_async_copy(ones_v, cnt_sh.at[dst_v.at[c]], csem).wait()
        pltpu.async_copy(ones_v, cnt_sh.at[dst_v.at[c]], csem, add=True)

    start_gather(0, 0, gsem0)
    if p == 0:
      pltpu.async_copy(zb, cnt_sh.at[dst_v.at[0]], csem, add=True)

    def pair(k, _):
      c0 = 2 * k
      start_gather(c0 + 1, 1, gsem1)
      finish(c0, 0, gsem0)
      start_gather(c0 + 2, 0, gsem0)
      finish(c0 + 1, 1, gsem1)
      return 0
    lax.fori_loop(0, (NCHUNK - 1) // 2, pair, 0)
    finish(NCHUNK - 1, 0, gsem0)
    if p == 0:
      pltpu.make_async_copy(ones_v, cnt_sh.at[dst_v.at[0]], csem).wait()

    plsc.subcore_barrier()

    pltpu.sync_copy(shared.at[pl.ds(base, WIN)],
                    msum_out.at[p].at[cid].at[pl.ds(base, WIN)])
    if p == 0:
      pltpu.sync_copy(cnt_sh.at[pl.ds(base, WIN)],
                      cnt_out.at[cid].at[pl.ds(base, WIN)])
      plsc.subcore_barrier()


def _edge_call(hr, src2, dst2, rat2):
  f = pl.kernel(
      _edge_body,
      out_type=(
          jax.ShapeDtypeStruct((2, NC, N, HD), jnp.float32),
          jax.ShapeDtypeStruct((NC, N, CW), jnp.float32),
      ),
      mesh=_sc_mesh(),
      compiler_params=pltpu.CompilerParams(use_tc_tiling_on_sc=False),
      scratch_types=[
          pltpu.VMEM((NCHUNK, CH), jnp.int32),
          pltpu.VMEM((NCHUNK, CH), jnp.int32),
          pltpu.VMEM((NCHUNK, CH), jnp.int32),
          pltpu.VMEM((2, NCHUNK, CH), jnp.int32),
          pltpu.VMEM((2, CH, HD), jnp.float32),
          pltpu.VMEM((CH, HD), jnp.float32),
          pltpu.VMEM((CH, CW), jnp.float32),
          pltpu.VMEM((CH, CW), jnp.float32),
          pltpu.VMEM_SHARED((N, HD), jnp.float32),
          pltpu.VMEM_SHARED((N, CW), jnp.float32),
          pltpu.SemaphoreType.DMA,
          pltpu.SemaphoreType.DMA,
          pltpu.SemaphoreType.DMA,
      ],
  )
  return f(hr, src2, dst2, rat2)


def _tail_body(msi_ref, cnti_ref, hi_ref, wd_i, wn_i, bl_i, pv, pvb,
               msu_ref, cntu_ref, hu_ref, wd_u, wn_u, bl_u, pw, pwb,
               item_out, user_out):
  def side(ms_ref, cnt_ref, hd_ref, wd, wn, bl, p, pb, out):
    ms = jnp.concatenate([ms_ref[0, 0] + ms_ref[0, 1],
                          ms_ref[1, 0] + ms_ref[1, 1]], axis=1)
    cnt = (cnt_ref[0] + cnt_ref[1])[:, :1]
    hn = jnp.where(cnt > 0, ms / jnp.maximum(cnt, 1.0), 0.0)
    acc = lax.dot_general(hd_ref[...], wd[...], (((1,), (1,)), ((), ())),
                          preferred_element_type=jnp.float32)
    acc += lax.dot_general(hn, wn[...], (((1,), (1,)), ((), ())),
                           preferred_element_type=jnp.float32)
    nh = jnp.maximum(acc + bl[...], 0.0)
    out[...] = lax.dot_general(nh, p[...], (((1,), (1,)), ((), ())),
                               preferred_element_type=jnp.float32) + pb[...]
  side(msi_ref, cnti_ref, hi_ref, wd_i, wn_i, bl_i, pv, pvb, item_out)
  side(msu_ref, cntu_ref, hu_ref, wd_u, wn_u, bl_u, pw, pwb, user_out)


def _tail_call(msum_i, cnt_i, h_item, wl_i, bl_i, vw, vb,
               msum_u, cnt_u, h_user, wl_u, bl_u, ww, wb):
  grid = (N // BM,)
  row_spec = pl.BlockSpec((BM, D), lambda m: (m, 0))
  ms_spec = pl.BlockSpec((2, NC, BM, HD), lambda m: (0, 0, m, 0))
  cnt_spec = pl.BlockSpec((NC, BM, CW), lambda m: (0, m, 0))
  w_spec = pl.BlockSpec((D, D), lambda m: (0, 0))
  b_spec = pl.BlockSpec((1, D), lambda m: (0, 0))
  args = (msum_i, cnt_i, h_item, wl_i[:, :D], wl_i[:, D:],
          bl_i.reshape(1, D), vw, vb.reshape(1, D),
          msum_u, cnt_u, h_user, wl_u[:, :D], wl_u[:, D:],
          bl_u.reshape(1, D), ww, wb.reshape(1, D))
  specs = [ms_spec, cnt_spec, row_spec, w_spec, w_spec, b_spec, w_spec,
           b_spec] * 2
  out_item, out_user = pl.pallas_call(
      _tail_body,
      grid=grid,
      in_specs=specs,
      out_specs=[row_spec, row_spec],
      out_shape=[
          jax.ShapeDtypeStruct((N, D), jnp.float32),
          jax.ShapeDtypeStruct((N, D), jnp.float32),
      ],
  )(*args)
  return out_item, out_user


def _pad_idx(idx):
  idx = idx.astype(jnp.int32)
  pad = NPAD - idx.shape[0]
  return jnp.pad(idx, (0, pad)).reshape(NW, GNC, GCH)


def kernel(user_ids, item_ids, user_gender, item_genres, edge_user,
           edge_item, edge_rating, user_table, item_table, gender_table,
           genre_table, Wr_watched, Wl_watched_w, Wl_watched_b,
           Wr_watchedby, Wl_watchedby_w, Wl_watchedby_b,
           W_w, W_b, V_w, V_b):
  hu_raw, hi_raw = _embed_call(user_table, _pad_idx(user_ids),
                               item_table, _pad_idx(item_ids))

  hr_w, hr_b, h_user, h_item = _msg_table_call(
      hu_raw[:N], user_gender, gender_table, Wr_watched,
      hi_raw[:N], item_genres, genre_table, Wr_watchedby)

  eu2 = edge_user.astype(jnp.int32).reshape(NW, NCHUNK, CH)
  ei2 = edge_item.astype(jnp.int32).reshape(NW, NCHUNK, CH)
  er2 = edge_rating.astype(jnp.int32).reshape(NW, NCHUNK, CH)

  msum_i, cnt_i = _edge_call(hr_w, eu2, ei2, er2)
  msum_u, cnt_u = _edge_call(hr_b, ei2, eu2, er2)

  out_item, out_user = _tail_call(
      msum_i, cnt_i, h_item, Wl_watched_w, Wl_watched_b, V_w, V_b,
      msum_u, cnt_u, h_user, Wl_watchedby_w, Wl_watchedby_b, W_w, W_b)
  return (out_user, out_item)


# speedup vs baseline: 1.3883x; 1.0985x over previous
import functools

import jax
import jax.numpy as jnp
from jax import lax
from jax.experimental import pallas as pl
from jax.experimental.pallas import tpu as pltpu
from jax.experimental.pallas import tpu_sc as plsc

N = 10000
E = 320000
D = 128
R = 6
NC, NS, LANES = 2, 16, 16
NW = NC * NS
EPT = E // NW
CH = 80
NCHUNK = EPT // CH
ROWS_PT = N // NS
WIN = 640
CW = 16
HD = 64
NPAD = 10240
GCH = 80
GPT = NPAD // NW
GNC = GPT // GCH
BM = 1000


def _sc_mesh():
  return plsc.VectorSubcoreMesh(
      core_axis_name="c", subcore_axis_name="s", num_cores=NC,
      num_subcores=NS)


def _embed_body(ut, uid2, it, iid2, hu_out, hi_out, idxv, rows, sem, osem):
  wid = lax.axis_index("s") * NC + lax.axis_index("c")
  tables = ((ut, uid2, hu_out), (it, iid2, hi_out))
  for t, (tbl, idx2, out) in enumerate(tables):
    pltpu.sync_copy(idx2.at[wid], idxv.at[t])
  gathers = []
  for t, (tbl, idx2, out) in enumerate(tables):
    for c in range(GNC):
      gathers.append(pltpu.async_copy(
          tbl.at[idxv.at[t].at[c]], rows.at[t].at[pl.ds(c * GCH, GCH)], sem))
  for g in gathers:
    g.wait()
  outcp = []
  for t, (tbl, idx2, out) in enumerate(tables):
    outcp.append(pltpu.async_copy(rows.at[t],
                                  out.at[pl.ds(wid * GPT, GPT)], osem))
  for o in outcp:
    o.wait()


def _embed_call(user_table, uid2, item_table, iid2):
  f = pl.kernel(
      _embed_body,
      out_type=tuple(jax.ShapeDtypeStruct((NPAD, D), jnp.float32)
                     for _ in range(2)),
      mesh=_sc_mesh(),
      compiler_params=pltpu.CompilerParams(use_tc_tiling_on_sc=False),
      scratch_types=[
          pltpu.VMEM((2, GNC, GCH), jnp.int32),
          pltpu.VMEM((2, GPT, D), jnp.float32),
          pltpu.SemaphoreType.DMA,
          pltpu.SemaphoreType.DMA,
      ],
  )
  return f(user_table, uid2, item_table, iid2)


def _msg_side(emb_ref, fid_ref, ftab_ref, wr_ref, hr_ref, h_ref):
  nfeat = ftab_ref.shape[0]
  onehot = (fid_ref[...] == lax.broadcasted_iota(jnp.int32, (1, nfeat), 1)
            ).astype(jnp.float32)
  feat = lax.dot_general(onehot, ftab_ref[...], (((1,), (0,)), ((), ())),
                         preferred_element_type=jnp.float32,
                         precision=lax.Precision.HIGHEST)
  h = emb_ref[...] + feat
  h_ref[...] = h
  for r in range(R):
    hr_ref[r] = lax.dot_general(h, wr_ref[r], (((1,), (1,)), ((), ())),
                                preferred_element_type=jnp.float32)


def _msg_table_body(embu_ref, fidu_ref, gtab_ref, wrw_ref,
                    embi_ref, fidi_ref, ntab_ref, wrb_ref,
                    hrw_ref, hrb_ref, hu_ref, hi_ref):
  _msg_side(embu_ref, fidu_ref, gtab_ref, wrw_ref, hrw_ref, hu_ref)
  _msg_side(embi_ref, fidi_ref, ntab_ref, wrb_ref, hrb_ref, hi_ref)


def _msg_table_call(embu, fidu, gtab, wrw, embi, fidi, ntab, wrb):
  grid = (N // BM,)
  row_spec = pl.BlockSpec((BM, D), lambda m: (m, 0))
  fid_spec = pl.BlockSpec((BM, 1), lambda m: (m, 0))
  wr_spec = pl.BlockSpec((R, D, D), lambda m: (0, 0, 0))
  hr_spec = pl.BlockSpec((R, BM, D), lambda m: (0, m, 0))
  hr_shape = jax.ShapeDtypeStruct((R, N, D), jnp.float32)
  hrw, hrb, hu, hi = pl.pallas_call(
      _msg_table_body,
      grid=grid,
      in_specs=[
          row_spec, fid_spec,
          pl.BlockSpec((gtab.shape[0], D), lambda m: (0, 0)), wr_spec,
          row_spec, fid_spec,
          pl.BlockSpec((ntab.shape[0], D), lambda m: (0, 0)), wr_spec,
      ],
      out_specs=[hr_spec, hr_spec, row_spec, row_spec],
      out_shape=[hr_shape, hr_shape,
                 jax.ShapeDtypeStruct((N, D), jnp.float32),
                 jax.ShapeDtypeStruct((N, D), jnp.float32)],
  )(embu, fidu.astype(jnp.int32).reshape(N, 1), gtab, wrw,
    embi, fidi.astype(jnp.int32).reshape(N, 1), ntab, wrb)
  return hrw.reshape(2 * R * N, HD), hrb.reshape(2 * R * N, HD), hu, hi


def _edge_body(hr, src2, dst2, rat2, msum_out, cnt_out,
               src_v, dst_v, rat_v, gix_v, rows, zrows, ones_v, zb,
               shared, cnt_sh, gsem0, gsem1, gsem2, gsem3,
               ssem0, ssem1, ssem2, ssem3, csem):
  cid = lax.axis_index("c")
  sid = lax.axis_index("s")
  wid = sid * NC + cid
  zero16 = jnp.zeros((LANES,), jnp.float32)
  ones16 = jnp.ones((LANES,), jnp.float32)

  pltpu.sync_copy(src2.at[wid], src_v)
  pltpu.sync_copy(dst2.at[wid], dst_v)
  pltpu.sync_copy(rat2.at[wid], rat_v)

  def gix_chunk(c, _):
    def gix_vec(j, _):
      g0 = (rat_v[c, pl.ds(j * LANES, LANES)] * N
            + src_v[c, pl.ds(j * LANES, LANES)]) * 2
      gix_v[0, c, pl.ds(j * LANES, LANES)] = g0
      gix_v[1, c, pl.ds(j * LANES, LANES)] = g0 + 1
      return 0
    return lax.fori_loop(0, CH // LANES, gix_vec, 0)
  lax.fori_loop(0, NCHUNK, gix_chunk, 0)

  def zrow(i, _):
    def zlane(j, _):
      zrows[i, pl.ds(j * LANES, LANES)] = zero16
      return 0
    return lax.fori_loop(0, HD // LANES, zlane, 0)
  lax.fori_loop(0, CH, zrow, 0)

  def fill16(i, _):
    ones_v[i, pl.ds(0, LANES)] = ones16
    zb[i, pl.ds(0, LANES)] = zero16
    return 0
  lax.fori_loop(0, CH, fill16, 0)

  base = sid * (ROWS_PT - 1)

  for p in range(2):
    for k in range(WIN // CH):
      pltpu.sync_copy(zrows, shared.at[pl.ds(base + k * CH, CH)])
      if p == 0:
        pltpu.sync_copy(zb, cnt_sh.at[pl.ds(base + k * CH, CH)])

    plsc.subcore_barrier()

    def start_gather(c, j, sem):
      pltpu.async_copy(hr.at[gix_v.at[p].at[c]], rows.at[j], sem)

    def wait_gather(c, j, sem):
      pltpu.make_async_copy(hr.at[gix_v.at[p].at[c]], rows.at[j], sem).wait()

    def issue_scatter(c, j, sem):
      pltpu.async_copy(rows.at[j], shared.at[dst_v.at[c]], sem, add=True)

    def wait_scatter(j, sem):
      pltpu.make_async_copy(rows.at[j], shared.at[dst_v.at[0]], sem).wait()

    def counts(c):
      if p == 0:
        pltpu.make_async_copy(ones_v, cnt_sh.at[dst_v.at[c]], csem).wait()
        pltpu.async_copy(ones_v, cnt_sh.at[dst_v.at[c]], csem, add=True)

    ssems = (ssem0, ssem1, ssem2, ssem3)
    gsems = (gsem0, gsem1, gsem2, gsem3)
    start_gather(0, 0, gsems[0])
    start_gather(1, 1, gsems[1])
    pltpu.async_copy(zrows, shared.at[dst_v.at[0]], ssems[2], add=True)
    pltpu.async_copy(zrows, shared.at[dst_v.at[0]], ssems[3], add=True)
    if p == 0:
      pltpu.async_copy(zb, cnt_sh.at[dst_v.at[0]], csem, add=True)

    def quad(k, _):
      for j in range(4):
        c = 4 * k + j
        wait_gather(c, j, gsems[j])
        issue_scatter(c, j, ssems[j])
        counts(c)
        j2 = (j + 2) % 4
        wait_scatter(j2, ssems[j2])

        @pl.when(c + 2 < NCHUNK)
        def _():
          start_gather(c + 2, j2, gsems[j2])
      return 0
    lax.fori_loop(0, NCHUNK // 4, quad, 0)
    cl = NCHUNK - 1
    wait_gather(cl, 0, gsems[0])
    wait_scatter(2, ssems[2])
    issue_scatter(cl, 0, ssems[0])
    counts(cl)
    wait_scatter(0, ssems[0])
    wait_scatter(3, ssems[3])
    if p == 0:
      pltpu.make_async_copy(ones_v, cnt_sh.at[dst_v.at[0]], csem).wait()

    plsc.subcore_barrier()

    pltpu.sync_copy(shared.at[pl.ds(base, WIN)],
                    msum_out.at[p].at[cid].at[pl.ds(base, WIN)])
    if p == 0:
      pltpu.sync_copy(cnt_sh.at[pl.ds(base, WIN)],
                      cnt_out.at[cid].at[pl.ds(base, WIN)])
      plsc.subcore_barrier()


def _edge_call(hr, src2, dst2, rat2):
  f = pl.kernel(
      _edge_body,
      out_type=(
          jax.ShapeDtypeStruct((2, NC, N, HD), jnp.float32),
          jax.ShapeDtypeStruct((NC, N, CW), jnp.float32),
      ),
      mesh=_sc_mesh(),
      compiler_params=pltpu.CompilerParams(use_tc_tiling_on_sc=False),
      scratch_types=[
          pltpu.VMEM((NCHUNK, CH), jnp.int32),
          pltpu.VMEM((NCHUNK, CH), jnp.int32),
          pltpu.VMEM((NCHUNK, CH), jnp.int32),
          pltpu.VMEM((2, NCHUNK, CH), jnp.int32),
          pltpu.VMEM((4, CH, HD), jnp.float32),
          pltpu.VMEM((CH, HD), jnp.float32),
          pltpu.VMEM((CH, CW), jnp.float32),
          pltpu.VMEM((CH, CW), jnp.float32),
          pltpu.VMEM_SHARED((N, HD), jnp.float32),
          pltpu.VMEM_SHARED((N, CW), jnp.float32),
      ] + [pltpu.SemaphoreType.DMA] * 9,
  )
  return f(hr, src2, dst2, rat2)


def _tail_body(msi_ref, cnti_ref, hi_ref, wd_i, wn_i, bl_i, pv, pvb,
               msu_ref, cntu_ref, hu_ref, wd_u, wn_u, bl_u, pw, pwb,
               item_out, user_out):
  def side(ms_ref, cnt_ref, hd_ref, wd, wn, bl, p, pb, out):
    ms = jnp.concatenate([ms_ref[0, 0] + ms_ref[0, 1],
                          ms_ref[1, 0] + ms_ref[1, 1]], axis=1)
    cnt = (cnt_ref[0] + cnt_ref[1])[:, :1]
    hn = jnp.where(cnt > 0, ms / jnp.maximum(cnt, 1.0), 0.0)
    acc = lax.dot_general(hd_ref[...], wd[...], (((1,), (1,)), ((), ())),
                          preferred_element_type=jnp.float32)
    acc += lax.dot_general(hn, wn[...], (((1,), (1,)), ((), ())),
                           preferred_element_type=jnp.float32)
    nh = jnp.maximum(acc + bl[...], 0.0)
    out[...] = lax.dot_general(nh, p[...], (((1,), (1,)), ((), ())),
                               preferred_element_type=jnp.float32) + pb[...]
  side(msi_ref, cnti_ref, hi_ref, wd_i, wn_i, bl_i, pv, pvb, item_out)
  side(msu_ref, cntu_ref, hu_ref, wd_u, wn_u, bl_u, pw, pwb, user_out)


def _tail_call(msum_i, cnt_i, h_item, wl_i, bl_i, vw, vb,
               msum_u, cnt_u, h_user, wl_u, bl_u, ww, wb):
  grid = (N // BM,)
  row_spec = pl.BlockSpec((BM, D), lambda m: (m, 0))
  ms_spec = pl.BlockSpec((2, NC, BM, HD), lambda m: (0, 0, m, 0))
  cnt_spec = pl.BlockSpec((NC, BM, CW), lambda m: (0, m, 0))
  w_spec = pl.BlockSpec((D, D), lambda m: (0, 0))
  b_spec = pl.BlockSpec((1, D), lambda m: (0, 0))
  args = (msum_i, cnt_i, h_item, wl_i[:, :D], wl_i[:, D:],
          bl_i.reshape(1, D), vw, vb.reshape(1, D),
          msum_u, cnt_u, h_user, wl_u[:, :D], wl_u[:, D:],
          bl_u.reshape(1, D), ww, wb.reshape(1, D))
  specs = [ms_spec, cnt_spec, row_spec, w_spec, w_spec, b_spec, w_spec,
           b_spec] * 2
  out_item, out_user = pl.pallas_call(
      _tail_body,
      grid=grid,
      in_specs=specs,
      out_specs=[row_spec, row_spec],
      out_shape=[
          jax.ShapeDtypeStruct((N, D), jnp.float32),
          jax.ShapeDtypeStruct((N, D), jnp.float32),
      ],
  )(*args)
  return out_item, out_user


def _pad_idx(idx):
  idx = idx.astype(jnp.int32)
  pad = NPAD - idx.shape[0]
  return jnp.pad(idx, (0, pad)).reshape(NW, GNC, GCH)


def kernel(user_ids, item_ids, user_gender, item_genres, edge_user,
           edge_item, edge_rating, user_table, item_table, gender_table,
           genre_table, Wr_watched, Wl_watched_w, Wl_watched_b,
           Wr_watchedby, Wl_watchedby_w, Wl_watchedby_b,
           W_w, W_b, V_w, V_b):
  hu_raw, hi_raw = _embed_call(user_table, _pad_idx(user_ids),
                               item_table, _pad_idx(item_ids))

  hr_w, hr_b, h_user, h_item = _msg_table_call(
      hu_raw[:N], user_gender, gender_table, Wr_watched,
      hi_raw[:N], item_genres, genre_table, Wr_watchedby)

  eu2 = edge_user.astype(jnp.int32).reshape(NW, NCHUNK, CH)
  ei2 = edge_item.astype(jnp.int32).reshape(NW, NCHUNK, CH)
  er2 = edge_rating.astype(jnp.int32).reshape(NW, NCHUNK, CH)

  msum_i, cnt_i = _edge_call(hr_w, eu2, ei2, er2)
  msum_u, cnt_u = _edge_call(hr_b, ei2, eu2, er2)

  out_item, out_user = _tail_call(
      msum_i, cnt_i, h_item, Wl_watched_w, Wl_watched_b, V_w, V_b,
      msum_u, cnt_u, h_user, Wl_watchedby_w, Wl_watchedby_b, W_w, W_b)
  return (out_user, out_item)
